# baseline (speedup 1.0000x reference)
import functools

import jax
import jax.numpy as jnp
from jax import lax
from jax.experimental import pallas as pl
from jax.experimental.pallas import tpu as pltpu
from jax.experimental.pallas import tpu_sc as plsc

OUT = 64
SEQ = 8192
BATCH = 16384

_info = plsc.get_sparse_core_info()
_NC, _NS = _info.num_cores, _info.num_subcores
_NW = _NC * _NS
_BPW = BATCH // _NW
_NCH = 4
_CH = _BPW // _NCH

_TBLK = 1024


def _table_body(pe_ref, w_ref, b_ref, sp_ref, t_ref):
    t = lax.dot_general(
        pe_ref[...], w_ref[...], (((1,), (1,)), ((), ())),
        preferred_element_type=jnp.float32,
    ) + b_ref[...]
    row = lax.broadcasted_iota(jnp.int32, (SEQ // 2, OUT), 0)
    is_first = pl.program_id(0) == 0
    t = jnp.where(jnp.logical_and(row == 0, is_first), sp_ref[...], t)
    t_ref[...] = jnp.concatenate([t, t], axis=1)


_build_table = pl.pallas_call(
    _table_body,
    grid=(2,),
    in_specs=[
        pl.BlockSpec((SEQ // 2, OUT), lambda i: (i, 0)),
        pl.BlockSpec((OUT, OUT), lambda i: (0, 0)),
        pl.BlockSpec((1, OUT), lambda i: (0, 0)),
        pl.BlockSpec((1, OUT), lambda i: (0, 0)),
    ],
    out_specs=pl.BlockSpec((SEQ // 2, 2 * OUT), lambda i: (i, 0)),
    out_shape=jax.ShapeDtypeStruct((SEQ, 2 * OUT), jnp.float32),
)

_mesh = plsc.VectorSubcoreMesh(core_axis_name="c", subcore_axis_name="s")


@functools.partial(
    pl.kernel,
    mesh=_mesh,
    out_type=jax.ShapeDtypeStruct((BATCH, OUT), jnp.float32),
    scratch_types=[
        pltpu.VMEM((_BPW,), jnp.int32),
        pltpu.VMEM((_CH, 2 * OUT), jnp.float32),
        pltpu.VMEM((_CH, 2 * OUT), jnp.float32),
        pltpu.VMEM((_CH, OUT), jnp.float32),
        pltpu.VMEM((_CH, OUT), jnp.float32),
        pltpu.SemaphoreType.DMA,
        pltpu.SemaphoreType.DMA,
    ],
)
def _gather_rows(table_hbm, idx_hbm, out_hbm, idx_v, rows0, rows1, st0, st1,
                 sem0, sem1):
    wid = lax.axis_index("s") * _NC + lax.axis_index("c")
    base = wid * _BPW
    pltpu.sync_copy(idx_hbm.at[pl.ds(base, _BPW)], idx_v)
    rows = (rows0, rows1)
    sts = (st0, st1)
    sems = (sem0, sem1)

    def gath(c):
        return pltpu.async_copy(
            table_hbm.at[idx_v.at[pl.ds(c * _CH, _CH)]], rows[c % 2], sems[c % 2])

    def compact(rows_v, st_v):
        def body(r, carry):
            for k in range(OUT // 16):
                st_v[r, pl.ds(k * 16, 16)] = rows_v[r, pl.ds(k * 16, 16)]
            return carry
        lax.fori_loop(0, _CH, body, 0)

    g = [gath(0), gath(1)]
    for c in range(_NCH):
        g[c].wait()
        compact(rows[c % 2], sts[c % 2])
        if c + 2 < _NCH:
            g.append(gath(c + 2))
        pltpu.sync_copy(sts[c % 2], out_hbm.at[pl.ds(base + c * _CH, _CH)])


def kernel(duration, special_table, pe, W, b):
    table = _build_table(pe, W, b.reshape(1, OUT), special_table)
    return _gather_rows(table, duration.astype(jnp.int32))


# speedup vs baseline: 1.0032x; 1.0032x over previous
import functools

import jax
import jax.numpy as jnp
from jax import lax
from jax.experimental import pallas as pl
from jax.experimental.pallas import tpu as pltpu
from jax.experimental.pallas import tpu_sc as plsc

OUT = 64
SEQ = 8192
BATCH = 16384

_info = plsc.get_sparse_core_info()
_NC, _NS = _info.num_cores, _info.num_subcores
_NW = _NC * _NS
_BPW = BATCH // _NW
_NCH = 4
_CH = _BPW // _NCH

_TBLK = 1024


def _table_body(pe_ref, w_ref, b_ref, sp_ref, t_ref):
    t = lax.dot_general(
        pe_ref[...], w_ref[...], (((1,), (1,)), ((), ())),
        preferred_element_type=jnp.float32,
    ) + b_ref[...]
    row = lax.broadcasted_iota(jnp.int32, (SEQ // 2, OUT), 0)
    is_first = pl.program_id(0) == 0
    t = jnp.where(jnp.logical_and(row == 0, is_first), sp_ref[...], t)
    t_ref[...] = jnp.concatenate([t, t], axis=1)


_build_table = pl.pallas_call(
    _table_body,
    grid=(2,),
    in_specs=[
        pl.BlockSpec((SEQ // 2, OUT), lambda i: (i, 0)),
        pl.BlockSpec((OUT, OUT), lambda i: (0, 0)),
        pl.BlockSpec((1, OUT), lambda i: (0, 0)),
        pl.BlockSpec((1, OUT), lambda i: (0, 0)),
    ],
    out_specs=pl.BlockSpec((SEQ // 2, 2 * OUT), lambda i: (i, 0)),
    out_shape=jax.ShapeDtypeStruct((SEQ, 2 * OUT), jnp.float32),
)

_mesh = plsc.VectorSubcoreMesh(core_axis_name="c", subcore_axis_name="s")


@functools.partial(
    pl.kernel,
    mesh=_mesh,
    out_type=jax.ShapeDtypeStruct((BATCH, OUT), jnp.float32),
    scratch_types=[
        pltpu.VMEM((_BPW,), jnp.int32),
        pltpu.VMEM((_CH, 2 * OUT), jnp.float32),
        pltpu.VMEM((_CH, 2 * OUT), jnp.float32),
        pltpu.VMEM((_CH, OUT), jnp.float32),
        pltpu.VMEM((_CH, OUT), jnp.float32),
        pltpu.SemaphoreType.DMA,
        pltpu.SemaphoreType.DMA,
    ],
)
def _gather_rows(table_hbm, idx_hbm, out_hbm, idx_v, rows0, rows1, st0, st1,
                 sem0, sem1):
    wid = lax.axis_index("s") * _NC + lax.axis_index("c")
    base = wid * _BPW
    pltpu.sync_copy(idx_hbm.at[pl.ds(base, _BPW)], idx_v)
    rows = (rows0, rows1)
    sts = (st0, st1)
    sems = (sem0, sem1)

    def gath(c):
        return pltpu.async_copy(
            table_hbm.at[idx_v.at[pl.ds(c * _CH, _CH)]], rows[c % 2], sems[c % 2])

    def compact(rows_v, st_v):
        @plsc.parallel_loop(0, _CH, unroll=8)
        def body(r):
            for k in range(OUT // 16):
                st_v[r, pl.ds(k * 16, 16)] = rows_v[r, pl.ds(k * 16, 16)]

    g = [gath(0), gath(1)]
    for c in range(_NCH):
        g[c].wait()
        compact(rows[c % 2], sts[c % 2])
        if c + 2 < _NCH:
            g.append(gath(c + 2))
        pltpu.sync_copy(sts[c % 2], out_hbm.at[pl.ds(base + c * _CH, _CH)])


def kernel(duration, special_table, pe, W, b):
    table = _build_table(pe, W, b.reshape(1, OUT), special_table)
    return _gather_rows(table, duration.astype(jnp.int32))


# speedup vs baseline: 1.0612x; 1.0579x over previous
import functools

import jax
import jax.numpy as jnp
from jax import lax
from jax.experimental import pallas as pl
from jax.experimental.pallas import tpu as pltpu
from jax.experimental.pallas import tpu_sc as plsc

OUT = 64
SEQ = 8192
BATCH = 16384

_info = plsc.get_sparse_core_info()
_NC, _NS = _info.num_cores, _info.num_subcores
_NW = _NC * _NS
_BPW = BATCH // _NW
_HPW = _BPW // 2


def _table_body(pe_ref, w_ref, b_ref, sp_ref, t_ref):
    t = lax.dot_general(
        pe_ref[...], w_ref[...], (((1,), (1,)), ((), ())),
        preferred_element_type=jnp.float32,
    ) + b_ref[...]
    row = lax.broadcasted_iota(jnp.int32, (SEQ // 2, OUT), 0)
    is_first = pl.program_id(0) == 0
    t = jnp.where(jnp.logical_and(row == 0, is_first), sp_ref[...], t)
    t_ref[...] = jnp.concatenate([t, t], axis=1)


_build_table = pl.pallas_call(
    _table_body,
    grid=(2,),
    in_specs=[
        pl.BlockSpec((SEQ // 2, OUT), lambda i: (i, 0)),
        pl.BlockSpec((OUT, OUT), lambda i: (0, 0)),
        pl.BlockSpec((1, OUT), lambda i: (0, 0)),
        pl.BlockSpec((1, OUT), lambda i: (0, 0)),
    ],
    out_specs=pl.BlockSpec((SEQ // 2, 2 * OUT), lambda i: (i, 0)),
    out_shape=jax.ShapeDtypeStruct((SEQ, 2 * OUT), jnp.float32),
)

_mesh = plsc.VectorSubcoreMesh(core_axis_name="c", subcore_axis_name="s")


@functools.partial(
    pl.kernel,
    mesh=_mesh,
    out_type=jax.ShapeDtypeStruct((BATCH, 2 * OUT), jnp.float32),
    scratch_types=[
        pltpu.VMEM((_BPW,), jnp.int32),
        pltpu.VMEM((_HPW, 2 * OUT), jnp.float32),
        pltpu.VMEM((_HPW, 2 * OUT), jnp.float32),
        pltpu.SemaphoreType.DMA,
        pltpu.SemaphoreType.DMA,
    ],
)
def _gather_rows(table_hbm, idx_hbm, out_hbm, idx_v, rows0, rows1, sem0, sem1):
    wid = lax.axis_index("s") * _NC + lax.axis_index("c")
    base = wid * _BPW
    pltpu.sync_copy(idx_hbm.at[pl.ds(base, _BPW)], idx_v)
    g0 = pltpu.async_copy(table_hbm.at[idx_v.at[pl.ds(0, _HPW)]], rows0, sem0)
    g1 = pltpu.async_copy(table_hbm.at[idx_v.at[pl.ds(_HPW, _HPW)]], rows1, sem1)
    g0.wait()
    pltpu.sync_copy(rows0, out_hbm.at[pl.ds(base, _HPW)])
    g1.wait()
    pltpu.sync_copy(rows1, out_hbm.at[pl.ds(base + _HPW, _HPW)])


def kernel(duration, special_table, pe, W, b):
    table = _build_table(pe, W, b.reshape(1, OUT), special_table)
    rows = _gather_rows(table, duration.astype(jnp.int32))
    return rows[:, :OUT]
